# baseline (speedup 1.0000x reference)
import functools

import jax
import jax.numpy as jnp
from jax import lax
from jax.experimental import pallas as pl
from jax.experimental.pallas import tpu as pltpu
from jax.experimental.pallas import tpu_sc as plsc

_N = 10000
_E = 320000
_D = 128
_NC = 2
_NS = 16
_NW = _NC * _NS
_EPW = _E // _NW
_CH = 80
_NCHUNK = _EPW // _CH
_NB = 5
_RPT = _N // _NS


def _sc_spmm(xb, src, dst, w):
    mesh = plsc.VectorSubcoreMesh(core_axis_name="c", subcore_axis_name="s")

    @functools.partial(
        pl.kernel,
        out_type=jax.ShapeDtypeStruct((_NC, _N, _D), jnp.bfloat16),
        mesh=mesh,
        scratch_types=[
            pltpu.VMEM((_NCHUNK, _CH), jnp.int32),
            pltpu.VMEM((_NCHUNK, _CH), jnp.int32),
            pltpu.VMEM((_NCHUNK, _CH), jnp.float32),
        ] + [pltpu.VMEM((_CH, _D), jnp.bfloat16)] * (2 * _NB)
          + [pltpu.VMEM_SHARED((_N, _D), jnp.bfloat16)]
          + [pltpu.SemaphoreType.DMA] * (2 * _NB),
        compiler_params=pltpu.CompilerParams(use_tc_tiling_on_sc=False,
                                             needs_layout_passes=False),
    )
    def k(x_hbm, src_hbm, dst_hbm, w_hbm, out_hbm, sidx, didx, wv, *bufs):
        rb = bufs[0:_NB]
        rf = bufs[_NB:2 * _NB]
        acc = bufs[2 * _NB]
        gsem = bufs[2 * _NB + 1:3 * _NB + 1]
        ssem = bufs[3 * _NB + 1:4 * _NB + 1]
        c = lax.axis_index("c")
        s = lax.axis_index("s")
        wid = s * _NC + c

        pltpu.sync_copy(src_hbm.at[wid], sidx)
        pltpu.sync_copy(dst_hbm.at[wid], didx)
        pltpu.sync_copy(w_hbm.at[wid], wv)

        zero32 = jnp.zeros((32,), jnp.bfloat16)

        def zrow(i, carry):
            for kk in range(_D // 32):
                rf[0][i, pl.ds(kk * 32, 32)] = zero32
            return carry

        lax.fori_loop(0, _CH, zrow, 0)
        for j in range(_RPT // _CH):
            pltpu.sync_copy(rf[0], acc.at[pl.ds(s * _RPT + j * _CH, _CH)])
        tail = _RPT - (_RPT // _CH) * _CH
        if tail:
            pltpu.sync_copy(
                rf[0].at[pl.ds(0, tail)],
                acc.at[pl.ds(s * _RPT + (_RPT // _CH) * _CH, tail)])
        plsc.subcore_barrier()

        def start_gather(cix, b):
            pltpu.async_copy(x_hbm.at[sidx.at[cix]], rb[b], gsem[b])

        def wait_gather(b):
            pltpu.make_async_copy(x_hbm.at[sidx.at[0]], rb[b],
                                  gsem[b]).wait()

        def scale(cix, b):
            @plsc.parallel_loop(0, _CH, 1, unroll=2)
            def _s(e):
                wb = plsc.load_gather(
                    wv, [jnp.full((16,), cix, jnp.int32),
                         jnp.full((16,), e, jnp.int32)])
                wb2 = plsc.pack(wb, wb, format=plsc.PackFormat.INTERLEAVED)
                for g in range(_D // 32):
                    sl = pl.ds(32 * g, 32)
                    rf[b][e, sl] = rb[b][e, sl] * wb2

        def start_scatter(cix, b):
            pltpu.async_copy(rf[b], acc.at[didx.at[cix]], ssem[b], add=True)

        def wait_scatter(b):
            pltpu.make_async_copy(rf[b], acc.at[didx.at[0]],
                                  ssem[b]).wait()

        for b in range(_NB):
            start_gather(b, b)
        for cc in range(_NB):
            wait_gather(cc)
            scale(cc, cc)
            start_scatter(cc, cc)
            start_gather(cc + _NB, cc)

        def group(i, carry):
            c0 = _NB * i
            for j in range(_NB):
                cix = c0 + j
                wait_gather(j)
                wait_scatter(j)
                scale(cix, j)
                start_scatter(cix, j)

                @pl.when(cix + _NB < _NCHUNK)
                def _():
                    start_gather(cix + _NB, j)
            return carry

        lax.fori_loop(1, _NCHUNK // _NB, group, 0)
        for b in range(_NB):
            wait_scatter(b)
        plsc.subcore_barrier()

        r0 = s * _RPT
        pltpu.sync_copy(acc.at[pl.ds(r0, _RPT)],
                        out_hbm.at[c, pl.ds(r0, _RPT)])

    return k(xb, src.reshape(_NW, _NCHUNK, _CH),
             dst.reshape(_NW, _NCHUNK, _CH), w.reshape(_NW, _NCHUNK, _CH))


def _combine_body(scal_ref, p_ref, x_ref, o_ref):
    w0 = scal_ref[0]
    w1 = scal_ref[1]
    b = scal_ref[2]
    agg = (p_ref[0].astype(jnp.float32) + p_ref[1].astype(jnp.float32))
    o_ref[...] = agg * w0 + x_ref[...] * w1 + b


def _combine(partials, x, scal):
    blk = 1000
    grid = (_N // blk,)
    return pl.pallas_call(
        _combine_body,
        grid=grid,
        in_specs=[
            pl.BlockSpec(memory_space=pltpu.SMEM),
            pl.BlockSpec((_NC, blk, _D), lambda i: (0, i, 0)),
            pl.BlockSpec((blk, _D), lambda i: (i, 0)),
        ],
        out_specs=pl.BlockSpec((blk, _D), lambda i: (i, 0)),
        out_shape=jax.ShapeDtypeStruct((_N, _D), jnp.float32),
    )(scal, partials, x)


def kernel(x, edge_index, edge_weight, conv_w, conv_b):
    dst = edge_index[0]
    src = edge_index[1]
    xb = x.astype(jnp.bfloat16)
    partials = _sc_spmm(xb, src, dst, edge_weight)
    scal = jnp.stack([conv_w[0, 0, 0, 0], conv_w[0, 0, 0, 1], conv_b[0]])
    return _combine(partials, x, scal)
